# baseline (device time: 24465 ns/iter reference)
import os

import jax
import jax.numpy as jnp
from jax import lax
from jax.experimental import pallas as pl
from jax.experimental.pallas import tpu as pltpu

_SKIP_COMM = os.environ.get("SKIP_COMM", "0") == "1"

N_DEV = 4
B_LOC = 2
SQ = 256
SKV = 256
HQ = 16
DH = 64
D_MODEL = 512
D_HEADS = HQ * DH
CHUNK = D_HEADS // N_DEV
H_PER = HQ // N_DEV
HALF_Q = D_MODEL // 2
HALF_O = CHUNK // 2
BLK = 64


def kernel(x, Wq, K_ext, V_ext, Wo):
    gb0 = lax.axis_index("i") * B_LOC
    k_loc = jnp.transpose(
        lax.dynamic_slice_in_dim(K_ext, gb0, B_LOC, axis=0)
        .astype(jnp.bfloat16), (0, 2, 1, 3))
    v_loc = jnp.transpose(
        lax.dynamic_slice_in_dim(V_ext, gb0, B_LOC, axis=0)
        .astype(jnp.bfloat16), (0, 2, 1, 3))

    def body(x_ref, wq_ref, k_ref, v_ref, wo_ref, out_ref,
             commq, commo, sendq, recvq, sendo, recvo):
        my = lax.axis_index("i")
        left = lax.rem(my + N_DEV - 1, N_DEV)
        right = lax.rem(my + 1, N_DEV)
        opp = lax.rem(my + 2, N_DEV)

        barrier = pltpu.get_barrier_semaphore()
        for nbr in (left, right):
            pl.semaphore_signal(barrier, inc=1, device_id=(nbr,),
                                device_id_type=pl.DeviceIdType.MESH)
        pl.semaphore_wait(barrier, 2)

        commq[0] = (wq_ref[...] * (0.125 * 1.4426950408889634)
                    ).astype(jnp.bfloat16)
        commo[0] = wo_ref[...].astype(jnp.bfloat16)

        def copies(src, dst, dev, idx):
            cq = pltpu.make_async_remote_copy(
                src_ref=src(commq), dst_ref=dst(commq),
                send_sem=sendq.at[idx], recv_sem=recvq.at[idx],
                device_id=(dev,), device_id_type=pl.DeviceIdType.MESH)
            co = pltpu.make_async_remote_copy(
                src_ref=src(commo), dst_ref=dst(commo),
                send_sem=sendo.at[idx], recv_sem=recvo.at[idx],
                device_id=(dev,), device_id_type=pl.DeviceIdType.MESH)
            return cq, co

        aL = copies(lambda c: c.at[0], lambda c: c.at[2], left, 0)
        aR = copies(lambda c: c.at[0], lambda c: c.at[1], right, 1)
        bR = copies(lambda c: c.at[1, 0:(HALF_Q if c is commq else HALF_O)],
                    lambda c: c.at[3, 0:(HALF_Q if c is commq else HALF_O)],
                    right, 2)
        bL = copies(lambda c: c.at[2, (HALF_Q if c is commq else HALF_O):],
                    lambda c: c.at[3, (HALF_Q if c is commq else HALF_O):],
                    left, 3)

        xb = [x_ref[b].astype(jnp.bfloat16) for b in range(B_LOC)]

        def attend(qh, kh, vh):
            q_o = jnp.concatenate([qh[0:BLK], qh[3 * BLK:]], axis=0)
            k_o = jnp.concatenate([kh[0:BLK], kh[3 * BLK:]], axis=0)
            v_o = jnp.concatenate([vh[0:BLK], vh[3 * BLK:]], axis=0)
            s_o = lax.dot_general(
                q_o, k_o, (((1,), (1,)), ((), ())),
                preferred_element_type=jnp.float32)
            w_o = jnp.exp2(s_o)
            r_o = 1.0 / jnp.sum(w_o, axis=1, keepdims=True)
            c_o = jnp.dot(w_o.astype(jnp.bfloat16), v_o,
                          preferred_element_type=jnp.float32) * r_o

            q_m = qh[BLK:3 * BLK]
            s_m = lax.dot_general(
                q_m, kh[0:3 * BLK], (((1,), (1,)), ((), ())),
                preferred_element_type=jnp.float32)
            w_m = jnp.exp2(s_m)
            r_m = 1.0 / jnp.sum(w_m, axis=1, keepdims=True)
            c_m = jnp.dot(w_m.astype(jnp.bfloat16), vh[0:3 * BLK],
                          preferred_element_type=jnp.float32) * r_m
            return jnp.concatenate([c_o[0:BLK], c_m, c_o[BLK:]], axis=0)

        def compute_chunk(slot, origin):
            wq_c = commq[slot]
            wo_c = commo[slot]
            for b in range(B_LOC):
                qc = jnp.dot(xb[b], wq_c,
                             preferred_element_type=jnp.float32)
                ctx_cols = []
                for j in range(H_PER):
                    hg = origin * H_PER + j
                    qh = qc[:, j * DH:(j + 1) * DH].astype(jnp.bfloat16)
                    kh = k_ref[b, pl.ds(hg, 1)].reshape(SKV, DH)
                    vh = v_ref[b, pl.ds(hg, 1)].reshape(SKV, DH)
                    ctx_cols.append(attend(qh, kh, vh))
                ctx = jnp.concatenate(ctx_cols, axis=1).astype(jnp.bfloat16)
                acc = jnp.dot(ctx, wo_c,
                              preferred_element_type=jnp.float32)
                if slot == 0:
                    out_ref[b] = acc
                else:
                    out_ref[b] = out_ref[b] + acc

        if not _SKIP_COMM:
            for c in aL + aR:
                c.start()
        compute_chunk(0, my)
        if not _SKIP_COMM:
            aR[0].wait_recv()
            aR[1].wait_recv()
            bR[0].start()
            bR[1].start()
            aL[0].wait_recv()
            aL[1].wait_recv()
            bL[0].start()
            bL[1].start()
        compute_chunk(1, left)
        compute_chunk(2, right)
        if not _SKIP_COMM:
            for c in bR + bL:
                c.wait_recv()
        compute_chunk(3, opp)
        if not _SKIP_COMM:
            for c in aL + aR + bR + bL:
                c.wait_send()

    return pl.pallas_call(
        body,
        out_shape=jax.ShapeDtypeStruct((B_LOC, SQ, D_MODEL), jnp.float32),
        in_specs=[pl.BlockSpec(memory_space=pltpu.VMEM)] * 5,
        out_specs=pl.BlockSpec(memory_space=pltpu.VMEM),
        scratch_shapes=[
            pltpu.VMEM((N_DEV, D_MODEL, CHUNK), jnp.bfloat16),
            pltpu.VMEM((N_DEV, CHUNK, D_MODEL), jnp.bfloat16),
            pltpu.SemaphoreType.DMA((4,)),
            pltpu.SemaphoreType.DMA((4,)),
            pltpu.SemaphoreType.DMA((4,)),
            pltpu.SemaphoreType.DMA((4,)),
        ],
        compiler_params=pltpu.CompilerParams(collective_id=0),
    )(x, Wq, k_loc, v_loc, Wo)


# device time: 24410 ns/iter; 1.0023x vs baseline; 1.0023x over previous
import jax
import jax.numpy as jnp
from jax import lax
from jax.experimental import pallas as pl
from jax.experimental.pallas import tpu as pltpu

N_DEV = 4
B_LOC = 2
SQ = 256
SKV = 256
HQ = 16
DH = 64
D_MODEL = 512
D_HEADS = HQ * DH
CHUNK = D_HEADS // N_DEV
H_PER = HQ // N_DEV
HALF_Q = D_MODEL // 2
HALF_O = CHUNK // 2
BLK = 64


def kernel(x, Wq, K_ext, V_ext, Wo):
    gb0 = lax.axis_index("i") * B_LOC
    k_loc = jnp.transpose(
        lax.dynamic_slice_in_dim(K_ext, gb0, B_LOC, axis=0)
        .astype(jnp.bfloat16), (0, 2, 1, 3))
    v_loc = jnp.transpose(
        lax.dynamic_slice_in_dim(V_ext, gb0, B_LOC, axis=0)
        .astype(jnp.bfloat16), (0, 2, 1, 3))

    def body(x_ref, wq_ref, k_ref, v_ref, wo_ref, out_ref,
             commq, commo, sendq, recvq, sendo, recvo):
        my = lax.axis_index("i")
        left = lax.rem(my + N_DEV - 1, N_DEV)
        right = lax.rem(my + 1, N_DEV)
        opp = lax.rem(my + 2, N_DEV)

        barrier = pltpu.get_barrier_semaphore()
        for nbr in (left, right):
            pl.semaphore_signal(barrier, inc=1, device_id=(nbr,),
                                device_id_type=pl.DeviceIdType.MESH)
        pl.semaphore_wait(barrier, 2)

        commq[0] = (wq_ref[...] * (0.125 * 1.4426950408889634)
                    ).astype(jnp.bfloat16)
        commo[0] = wo_ref[...].astype(jnp.bfloat16)

        def copies(src, dst, dev, idx):
            cq = pltpu.make_async_remote_copy(
                src_ref=src(commq), dst_ref=dst(commq),
                send_sem=sendq.at[idx], recv_sem=recvq.at[idx],
                device_id=(dev,), device_id_type=pl.DeviceIdType.MESH)
            co = pltpu.make_async_remote_copy(
                src_ref=src(commo), dst_ref=dst(commo),
                send_sem=sendo.at[idx], recv_sem=recvo.at[idx],
                device_id=(dev,), device_id_type=pl.DeviceIdType.MESH)
            return cq, co

        aL = copies(lambda c: c.at[0], lambda c: c.at[2], left, 0)
        aR = copies(lambda c: c.at[0], lambda c: c.at[1], right, 1)
        bR = copies(lambda c: c.at[1, 0:(HALF_Q if c is commq else HALF_O)],
                    lambda c: c.at[3, 0:(HALF_Q if c is commq else HALF_O)],
                    right, 2)
        bL = copies(lambda c: c.at[2, (HALF_Q if c is commq else HALF_O):],
                    lambda c: c.at[3, (HALF_Q if c is commq else HALF_O):],
                    left, 3)

        xb = [x_ref[b].astype(jnp.bfloat16) for b in range(B_LOC)]

        def attend(qh, kh, vh):
            q_o = jnp.concatenate([qh[0:BLK], qh[3 * BLK:]], axis=0)
            k_o = jnp.concatenate([kh[0:BLK], kh[3 * BLK:]], axis=0)
            v_o = jnp.concatenate([vh[0:BLK], vh[3 * BLK:]], axis=0)
            s_o = lax.dot_general(
                q_o, k_o, (((1,), (1,)), ((), ())),
                preferred_element_type=jnp.float32)
            w_o = jnp.exp2(s_o)
            r_o = 1.0 / jnp.sum(w_o, axis=1, keepdims=True)
            c_o = jnp.dot(w_o.astype(jnp.bfloat16), v_o,
                          preferred_element_type=jnp.float32) * r_o

            q_m = qh[BLK:3 * BLK]
            s_m = lax.dot_general(
                q_m, kh[0:3 * BLK], (((1,), (1,)), ((), ())),
                preferred_element_type=jnp.float32)
            w_m = jnp.exp2(s_m)
            r_m = 1.0 / jnp.sum(w_m, axis=1, keepdims=True)
            c_m = jnp.dot(w_m.astype(jnp.bfloat16), vh[0:3 * BLK],
                          preferred_element_type=jnp.float32) * r_m
            return jnp.concatenate([c_o[0:BLK], c_m, c_o[BLK:]], axis=0)

        def compute_chunk(slot, origin):
            wq_c = commq[slot]
            wo_c = commo[slot]
            for b in range(B_LOC):
                qc = jnp.dot(xb[b], wq_c,
                             preferred_element_type=jnp.float32)
                ctx_cols = []
                for j in range(H_PER):
                    hg = origin * H_PER + j
                    qh = qc[:, j * DH:(j + 1) * DH].astype(jnp.bfloat16)
                    kh = k_ref[b, pl.ds(hg, 1)].reshape(SKV, DH)
                    vh = v_ref[b, pl.ds(hg, 1)].reshape(SKV, DH)
                    ctx_cols.append(attend(qh, kh, vh))
                ctx = jnp.concatenate(ctx_cols, axis=1).astype(jnp.bfloat16)
                acc = jnp.dot(ctx, wo_c,
                              preferred_element_type=jnp.float32)
                if slot == 0:
                    out_ref[b] = acc
                else:
                    out_ref[b] = out_ref[b] + acc

        for c in aL + aR:
            c.start()
        compute_chunk(0, my)
        aR[0].wait_recv()
        aR[1].wait_recv()
        bR[0].start()
        bR[1].start()
        aL[0].wait_recv()
        aL[1].wait_recv()
        bL[0].start()
        bL[1].start()
        compute_chunk(1, left)
        compute_chunk(2, right)
        for c in bR + bL:
            c.wait_recv()
        compute_chunk(3, opp)
        for c in aL + aR + bR + bL:
            c.wait_send()

    return pl.pallas_call(
        body,
        out_shape=jax.ShapeDtypeStruct((B_LOC, SQ, D_MODEL), jnp.float32),
        in_specs=[pl.BlockSpec(memory_space=pltpu.VMEM)] * 5,
        out_specs=pl.BlockSpec(memory_space=pltpu.VMEM),
        scratch_shapes=[
            pltpu.VMEM((N_DEV, D_MODEL, CHUNK), jnp.bfloat16),
            pltpu.VMEM((N_DEV, CHUNK, D_MODEL), jnp.bfloat16),
            pltpu.SemaphoreType.DMA((4,)),
            pltpu.SemaphoreType.DMA((4,)),
            pltpu.SemaphoreType.DMA((4,)),
            pltpu.SemaphoreType.DMA((4,)),
        ],
        compiler_params=pltpu.CompilerParams(collective_id=0),
    )(x, Wq, k_loc, v_loc, Wo)
